# Initial kernel scaffold; baseline (speedup 1.0000x reference)
import jax
import jax.numpy as jnp
from jax.experimental import pallas as pl


def kernel(weight, IDX, G):
    raise NotImplementedError("write your pallas kernel here")


# speedup vs baseline: 361.3367x; 361.3367x over previous
import functools

import jax
import jax.numpy as jnp
from jax import lax
from jax.experimental import pallas as pl
from jax.experimental.pallas import tpu as pltpu
from jax.experimental.pallas import tpu_sc as plsc

_WSIZE = 1280000
_NROW, _NCOL = 100000, 128
_N = _NROW * _NCOL
_NC, _NS = 2, 16
_NW = _NC * _NS
_PER_W = _N // _NW
_CHUNK = 16000
_NCHUNK = _PER_W // _CHUNK
_VECS = _CHUNK // 16


def _roast_body(w_hbm, idx_hbm, g_hbm, out_hbm, idx_v, g_v, val_v, table, sem):
    cid = lax.axis_index("c")
    sid = lax.axis_index("s")
    wid = sid * _NC + cid

    seg = _WSIZE // _NS
    pltpu.sync_copy(
        w_hbm.at[pl.ds(sid * seg, seg)], table.at[pl.ds(sid * seg, seg)]
    )
    plsc.subcore_barrier()

    def chunk_body(k, carry):
        base = wid * _PER_W + k * _CHUNK
        pltpu.sync_copy(idx_hbm.at[pl.ds(base, _CHUNK)], idx_v)
        pltpu.sync_copy(g_hbm.at[pl.ds(base, _CHUNK)], g_v)
        pltpu.async_copy(table.at[idx_v], val_v, sem).wait()

        def mul_body(i, c):
            s = pl.ds(i * 16, 16)
            val_v[s] = val_v[s] * g_v[s]
            return c

        lax.fori_loop(0, _VECS, mul_body, 0, unroll=8)
        pltpu.sync_copy(val_v, out_hbm.at[pl.ds(base, _CHUNK)])
        return carry

    lax.fori_loop(0, _NCHUNK, chunk_body, 0)


def kernel(weight, IDX, G):
    mesh = plsc.VectorSubcoreMesh(
        core_axis_name="c", subcore_axis_name="s", num_cores=_NC,
        num_subcores=_NS,
    )
    roast = pl.kernel(
        _roast_body,
        out_type=jax.ShapeDtypeStruct((_N,), jnp.float32),
        mesh=mesh,
        scratch_types=[
            pltpu.VMEM((_CHUNK,), jnp.int32),
            pltpu.VMEM((_CHUNK,), jnp.float32),
            pltpu.VMEM((_CHUNK,), jnp.float32),
            pltpu.VMEM_SHARED((_WSIZE,), jnp.float32),
            pltpu.SemaphoreType.DMA,
        ],
    )
    out = roast(weight, IDX.reshape(-1), G.reshape(-1))
    return out.reshape(_NROW, _NCOL)


# speedup vs baseline: 467.1059x; 1.2927x over previous
import functools

import jax
import jax.numpy as jnp
from jax import lax
from jax.experimental import pallas as pl
from jax.experimental.pallas import tpu as pltpu
from jax.experimental.pallas import tpu_sc as plsc

_WSIZE = 1280000
_NROW, _NCOL = 100000, 128
_N = _NROW * _NCOL
_NC, _NS = 2, 16
_NW = _NC * _NS
_PER_W = _N // _NW
_CHUNK = 8000
_NCHUNK = _PER_W // _CHUNK
_PAIRS = _NCHUNK // 2
_VECS = _CHUNK // 16


def _roast_body(w_hbm, idx_hbm, g_hbm, out_hbm,
                idx0, g0, val0, idx1, g1, val1, table,
                sin0, sin1, sg0, sg1, so0, so1):
    cid = lax.axis_index("c")
    sid = lax.axis_index("s")
    wid = sid * _NC + cid
    w0 = wid * _PER_W

    seg = _WSIZE // _NS
    pltpu.sync_copy(
        w_hbm.at[pl.ds(sid * seg, seg)], table.at[pl.ds(sid * seg, seg)]
    )
    plsc.subcore_barrier()

    def issue_in(k, idx_v, g_v, sem):
        base = w0 + k * _CHUNK
        pltpu.async_copy(idx_hbm.at[pl.ds(base, _CHUNK)], idx_v, sem)
        pltpu.async_copy(g_hbm.at[pl.ds(base, _CHUNK)], g_v, sem)

    def wait_in(k, idx_v, g_v, sem):
        base = w0 + k * _CHUNK
        pltpu.make_async_copy(idx_hbm.at[pl.ds(base, _CHUNK)], idx_v, sem).wait()
        pltpu.make_async_copy(g_hbm.at[pl.ds(base, _CHUNK)], g_v, sem).wait()

    def issue_out(k, val_v, sem):
        base = w0 + k * _CHUNK
        pltpu.async_copy(val_v, out_hbm.at[pl.ds(base, _CHUNK)], sem)

    def wait_out(k, val_v, sem):
        base = w0 + k * _CHUNK
        pltpu.make_async_copy(val_v, out_hbm.at[pl.ds(base, _CHUNK)], sem).wait()

    def multiply(val_v, g_v):
        def mul_body(i, c):
            s = pl.ds(i * 16, 16)
            val_v[s] = val_v[s] * g_v[s]
            return c

        lax.fori_loop(0, _VECS, mul_body, 0, unroll=8)

    issue_in(0, idx0, g0, sin0)
    issue_in(1, idx1, g1, sin1)
    wait_in(0, idx0, g0, sin0)
    pltpu.async_copy(table.at[idx0], val0, sg0)

    def pair_body(i, carry):
        a = 2 * i
        b = a + 1

        pltpu.make_async_copy(table.at[idx0], val0, sg0).wait()
        multiply(val0, g0)
        issue_out(a, val0, so0)

        @pl.when(i < _PAIRS - 1)
        def _():
            issue_in(a + 2, idx0, g0, sin0)

        wait_in(b, idx1, g1, sin1)

        @pl.when(i > 0)
        def _():
            wait_out(b - 2, val1, so1)

        pltpu.async_copy(table.at[idx1], val1, sg1)

        pltpu.make_async_copy(table.at[idx1], val1, sg1).wait()
        multiply(val1, g1)
        issue_out(b, val1, so1)

        @pl.when(i < _PAIRS - 1)
        def _():
            issue_in(b + 2, idx1, g1, sin1)
            wait_in(a + 2, idx0, g0, sin0)
            wait_out(a, val0, so0)
            pltpu.async_copy(table.at[idx0], val0, sg0)

        return carry

    lax.fori_loop(0, _PAIRS, pair_body, 0)

    wait_out(_NCHUNK - 2, val0, so0)
    wait_out(_NCHUNK - 1, val1, so1)


def kernel(weight, IDX, G):
    mesh = plsc.VectorSubcoreMesh(
        core_axis_name="c", subcore_axis_name="s", num_cores=_NC,
        num_subcores=_NS,
    )
    roast = pl.kernel(
        _roast_body,
        out_type=jax.ShapeDtypeStruct((_N,), jnp.float32),
        mesh=mesh,
        scratch_types=[
            pltpu.VMEM((_CHUNK,), jnp.int32),
            pltpu.VMEM((_CHUNK,), jnp.float32),
            pltpu.VMEM((_CHUNK,), jnp.float32),
            pltpu.VMEM((_CHUNK,), jnp.int32),
            pltpu.VMEM((_CHUNK,), jnp.float32),
            pltpu.VMEM((_CHUNK,), jnp.float32),
            pltpu.VMEM_SHARED((_WSIZE,), jnp.float32),
            pltpu.SemaphoreType.DMA,
            pltpu.SemaphoreType.DMA,
            pltpu.SemaphoreType.DMA,
            pltpu.SemaphoreType.DMA,
            pltpu.SemaphoreType.DMA,
            pltpu.SemaphoreType.DMA,
        ],
    )
    out = roast(weight, IDX.reshape(-1), G.reshape(-1))
    return out.reshape(_NROW, _NCOL)


# speedup vs baseline: 571.5414x; 1.2236x over previous
import functools

import jax
import jax.numpy as jnp
from jax import lax
from jax.experimental import pallas as pl
from jax.experimental.pallas import tpu as pltpu
from jax.experimental.pallas import tpu_sc as plsc

_WSIZE = 1280000
_NROW, _NCOL = 100000, 128
_N = _NROW * _NCOL
_NC, _NS = 2, 16
_NW = _NC * _NS
_PER_W = _N // _NW
_CHUNK = 8000
_NCHUNK = _PER_W // _CHUNK
_PAIRS = _NCHUNK // 2
_VECS = _CHUNK // 16


def _roast_body(w_hbm, idx_hbm, g_hbm, out_hbm,
                idx0, g0, val0, idx1, g1, val1, table,
                sin0, sin1, sg0, sg1, so0, so1):
    cid = lax.axis_index("c")
    sid = lax.axis_index("s")
    wid = sid * _NC + cid
    w0 = wid * _PER_W

    seg = _WSIZE // _NS
    pltpu.sync_copy(
        w_hbm.at[pl.ds(sid * seg, seg)], table.at[pl.ds(sid * seg, seg)]
    )
    plsc.subcore_barrier()

    def issue_in(k, idx_v, g_v, sem):
        base = w0 + k * _CHUNK
        pltpu.async_copy(idx_hbm.at[pl.ds(base, _CHUNK)], idx_v, sem)
        pltpu.async_copy(g_hbm.at[pl.ds(base, _CHUNK)], g_v, sem)

    def wait_in(k, idx_v, g_v, sem):
        base = w0 + k * _CHUNK
        pltpu.make_async_copy(idx_hbm.at[pl.ds(base, _CHUNK)], idx_v, sem).wait()
        pltpu.make_async_copy(g_hbm.at[pl.ds(base, _CHUNK)], g_v, sem).wait()

    def issue_out(k, val_v, sem):
        base = w0 + k * _CHUNK
        pltpu.async_copy(val_v, out_hbm.at[pl.ds(base, _CHUNK)], sem)

    def wait_out(k, val_v, sem):
        base = w0 + k * _CHUNK
        pltpu.make_async_copy(val_v, out_hbm.at[pl.ds(base, _CHUNK)], sem).wait()

    def multiply(val_v, g_v):
        def mul_body(i, c):
            s = pl.ds(i * 16, 16)
            val_v[s] = val_v[s] * g_v[s]
            return c

        lax.fori_loop(0, _VECS, mul_body, 0, unroll=8)

    issue_in(0, idx0, g0, sin0)
    issue_in(1, idx1, g1, sin1)
    wait_in(0, idx0, g0, sin0)
    pltpu.async_copy(table.at[idx0], val0, sg0)

    def pair_body(i, carry):
        a = 2 * i
        b = a + 1

        pltpu.make_async_copy(table.at[idx0], val0, sg0).wait()

        wait_in(b, idx1, g1, sin1)

        @pl.when(i > 0)
        def _():
            wait_out(b - 2, val1, so1)

        pltpu.async_copy(table.at[idx1], val1, sg1)

        multiply(val0, g0)
        issue_out(a, val0, so0)

        @pl.when(i < _PAIRS - 1)
        def _():
            issue_in(a + 2, idx0, g0, sin0)

        pltpu.make_async_copy(table.at[idx1], val1, sg1).wait()

        @pl.when(i < _PAIRS - 1)
        def _():
            wait_in(a + 2, idx0, g0, sin0)
            wait_out(a, val0, so0)
            pltpu.async_copy(table.at[idx0], val0, sg0)

        multiply(val1, g1)
        issue_out(b, val1, so1)

        @pl.when(i < _PAIRS - 1)
        def _():
            issue_in(b + 2, idx1, g1, sin1)

        return carry

    lax.fori_loop(0, _PAIRS, pair_body, 0)

    wait_out(_NCHUNK - 2, val0, so0)
    wait_out(_NCHUNK - 1, val1, so1)


def kernel(weight, IDX, G):
    mesh = plsc.VectorSubcoreMesh(
        core_axis_name="c", subcore_axis_name="s", num_cores=_NC,
        num_subcores=_NS,
    )
    roast = pl.kernel(
        _roast_body,
        out_type=jax.ShapeDtypeStruct((_N,), jnp.float32),
        mesh=mesh,
        scratch_types=[
            pltpu.VMEM((_CHUNK,), jnp.int32),
            pltpu.VMEM((_CHUNK,), jnp.float32),
            pltpu.VMEM((_CHUNK,), jnp.float32),
            pltpu.VMEM((_CHUNK,), jnp.int32),
            pltpu.VMEM((_CHUNK,), jnp.float32),
            pltpu.VMEM((_CHUNK,), jnp.float32),
            pltpu.VMEM_SHARED((_WSIZE,), jnp.float32),
            pltpu.SemaphoreType.DMA,
            pltpu.SemaphoreType.DMA,
            pltpu.SemaphoreType.DMA,
            pltpu.SemaphoreType.DMA,
            pltpu.SemaphoreType.DMA,
            pltpu.SemaphoreType.DMA,
        ],
    )
    out = roast(weight, IDX.reshape(-1), G.reshape(-1))
    return out.reshape(_NROW, _NCOL)


# speedup vs baseline: 820.4944x; 1.4356x over previous
import functools

import jax
import jax.numpy as jnp
from jax import lax
from jax.experimental import pallas as pl
from jax.experimental.pallas import tpu as pltpu
from jax.experimental.pallas import tpu_sc as plsc

_WSIZE = 1280000
_NROW, _NCOL = 100000, 128
_N = _NROW * _NCOL
_NC, _NS = 2, 16
_NW = _NC * _NS
_PER_W = _N // _NW
_CHUNK = 8000
_NCHUNK = _PER_W // _CHUNK
_PAIRS = _NCHUNK // 2
_VECS = _CHUNK // 16


def _roast_body(w_hbm, idx_hbm, g_hbm, out_hbm,
                idx0, g0, val0, idx1, g1, val1, table,
                sin0, sin1, sg0, sg1, so0, so1):
    cid = lax.axis_index("c")
    sid = lax.axis_index("s")
    wid = sid * _NC + cid
    w0 = wid * _PER_W

    seg = _WSIZE // _NS
    pltpu.sync_copy(
        w_hbm.at[pl.ds(sid * seg, seg)], table.at[pl.ds(sid * seg, seg)]
    )
    plsc.subcore_barrier()

    def issue_in(k, idx_v, g_v, sem):
        base = w0 + k * _CHUNK
        pltpu.async_copy(idx_hbm.at[pl.ds(base, _CHUNK)], idx_v, sem)
        pltpu.async_copy(g_hbm.at[pl.ds(base, _CHUNK)], g_v, sem)

    def wait_in(k, idx_v, g_v, sem):
        base = w0 + k * _CHUNK
        pltpu.make_async_copy(idx_hbm.at[pl.ds(base, _CHUNK)], idx_v, sem).wait()
        pltpu.make_async_copy(g_hbm.at[pl.ds(base, _CHUNK)], g_v, sem).wait()

    def issue_out(k, val_v, sem):
        base = w0 + k * _CHUNK
        pltpu.async_copy(val_v, out_hbm.at[pl.ds(base, _CHUNK)], sem)

    def wait_out(k, val_v, sem):
        base = w0 + k * _CHUNK
        pltpu.make_async_copy(val_v, out_hbm.at[pl.ds(base, _CHUNK)], sem).wait()

    def multiply(val_v, g_v):
        @plsc.parallel_loop(0, _CHUNK, 16, unroll=8)
        def _(i):
            s = pl.ds(i, 16)
            val_v[s] = val_v[s] * g_v[s]

    issue_in(0, idx0, g0, sin0)
    issue_in(1, idx1, g1, sin1)
    wait_in(0, idx0, g0, sin0)
    pltpu.async_copy(table.at[idx0], val0, sg0)

    def pair_body(i, carry):
        a = 2 * i
        b = a + 1

        pltpu.make_async_copy(table.at[idx0], val0, sg0).wait()

        wait_in(b, idx1, g1, sin1)

        @pl.when(i > 0)
        def _():
            wait_out(b - 2, val1, so1)

        pltpu.async_copy(table.at[idx1], val1, sg1)

        multiply(val0, g0)
        issue_out(a, val0, so0)

        @pl.when(i < _PAIRS - 1)
        def _():
            issue_in(a + 2, idx0, g0, sin0)

        pltpu.make_async_copy(table.at[idx1], val1, sg1).wait()

        @pl.when(i < _PAIRS - 1)
        def _():
            wait_in(a + 2, idx0, g0, sin0)
            wait_out(a, val0, so0)
            pltpu.async_copy(table.at[idx0], val0, sg0)

        multiply(val1, g1)
        issue_out(b, val1, so1)

        @pl.when(i < _PAIRS - 1)
        def _():
            issue_in(b + 2, idx1, g1, sin1)

        return carry

    lax.fori_loop(0, _PAIRS, pair_body, 0)

    wait_out(_NCHUNK - 2, val0, so0)
    wait_out(_NCHUNK - 1, val1, so1)


def kernel(weight, IDX, G):
    mesh = plsc.VectorSubcoreMesh(
        core_axis_name="c", subcore_axis_name="s", num_cores=_NC,
        num_subcores=_NS,
    )
    roast = pl.kernel(
        _roast_body,
        out_type=jax.ShapeDtypeStruct((_N,), jnp.float32),
        mesh=mesh,
        scratch_types=[
            pltpu.VMEM((_CHUNK,), jnp.int32),
            pltpu.VMEM((_CHUNK,), jnp.float32),
            pltpu.VMEM((_CHUNK,), jnp.float32),
            pltpu.VMEM((_CHUNK,), jnp.int32),
            pltpu.VMEM((_CHUNK,), jnp.float32),
            pltpu.VMEM((_CHUNK,), jnp.float32),
            pltpu.VMEM_SHARED((_WSIZE,), jnp.float32),
            pltpu.SemaphoreType.DMA,
            pltpu.SemaphoreType.DMA,
            pltpu.SemaphoreType.DMA,
            pltpu.SemaphoreType.DMA,
            pltpu.SemaphoreType.DMA,
            pltpu.SemaphoreType.DMA,
        ],
    )
    out = roast(weight, IDX.reshape(-1), G.reshape(-1))
    return out.reshape(_NROW, _NCOL)


# speedup vs baseline: 854.9857x; 1.0420x over previous
import functools

import jax
import jax.numpy as jnp
from jax import lax
from jax.experimental import pallas as pl
from jax.experimental.pallas import tpu as pltpu
from jax.experimental.pallas import tpu_sc as plsc

_WSIZE = 1280000
_NROW, _NCOL = 100000, 128
_N = _NROW * _NCOL
_NC, _NS = 2, 16
_NW = _NC * _NS
_PER_W = _N // _NW
_NBUF = 4
_CHUNK = 4000
_NCHUNK = _PER_W // _CHUNK
_QUADS = _NCHUNK // _NBUF


def _roast_body(w_hbm, idx_hbm, g_hbm, out_hbm, *scratch):
    idx = scratch[0:_NBUF]
    g = scratch[_NBUF:2 * _NBUF]
    val = scratch[2 * _NBUF:3 * _NBUF]
    table = scratch[3 * _NBUF]
    sin = scratch[3 * _NBUF + 1:3 * _NBUF + 1 + _NBUF]
    sg = scratch[3 * _NBUF + 1 + _NBUF:3 * _NBUF + 1 + 2 * _NBUF]
    so = scratch[3 * _NBUF + 1 + 2 * _NBUF:3 * _NBUF + 1 + 3 * _NBUF]

    cid = lax.axis_index("c")
    sid = lax.axis_index("s")
    wid = sid * _NC + cid
    w0 = wid * _PER_W

    seg = _WSIZE // _NS
    pltpu.sync_copy(
        w_hbm.at[pl.ds(sid * seg, seg)], table.at[pl.ds(sid * seg, seg)]
    )
    plsc.subcore_barrier()

    def issue_in(k, s):
        base = w0 + k * _CHUNK
        pltpu.async_copy(idx_hbm.at[pl.ds(base, _CHUNK)], idx[s], sin[s])
        pltpu.async_copy(g_hbm.at[pl.ds(base, _CHUNK)], g[s], sin[s])

    def wait_in(k, s):
        base = w0 + k * _CHUNK
        pltpu.make_async_copy(idx_hbm.at[pl.ds(base, _CHUNK)], idx[s], sin[s]).wait()
        pltpu.make_async_copy(g_hbm.at[pl.ds(base, _CHUNK)], g[s], sin[s]).wait()

    def issue_out(k, s):
        base = w0 + k * _CHUNK
        pltpu.async_copy(val[s], out_hbm.at[pl.ds(base, _CHUNK)], so[s])

    def wait_out(k, s):
        base = w0 + k * _CHUNK
        pltpu.make_async_copy(val[s], out_hbm.at[pl.ds(base, _CHUNK)], so[s]).wait()

    def issue_gather(s):
        pltpu.async_copy(table.at[idx[s]], val[s], sg[s])

    def wait_gather(s):
        pltpu.make_async_copy(table.at[idx[s]], val[s], sg[s]).wait()

    def multiply(s):
        val_v, g_v = val[s], g[s]

        @plsc.parallel_loop(0, _CHUNK, 16, unroll=8)
        def _(i):
            sl = pl.ds(i, 16)
            val_v[sl] = val_v[sl] * g_v[sl]

    for s in range(_NBUF):
        issue_in(s, s)
    wait_in(0, 0)
    issue_gather(0)
    wait_in(1, 1)
    issue_gather(1)

    def quad_body(i, carry):
        for j in range(_NBUF):
            s = j
            k = _NBUF * i + j

            wait_gather(s)

            s2 = (j + 2) % _NBUF
            if j < 2:
                wait_in(k + 2, s2)

                @pl.when(i > 0)
                def _():
                    wait_out(k + 2 - _NBUF, s2)

                issue_gather(s2)
            else:

                @pl.when(i < _QUADS - 1)
                def _():
                    wait_in(k + 2, s2)
                    wait_out(k + 2 - _NBUF, s2)
                    issue_gather(s2)

            multiply(s)
            issue_out(k, s)

            @pl.when(i < _QUADS - 1)
            def _():
                issue_in(k + _NBUF, s)

        return carry

    lax.fori_loop(0, _QUADS, quad_body, 0)

    wait_out(_NCHUNK - 4, 0)
    wait_out(_NCHUNK - 3, 1)
    wait_out(_NCHUNK - 2, 2)
    wait_out(_NCHUNK - 1, 3)


def kernel(weight, IDX, G):
    mesh = plsc.VectorSubcoreMesh(
        core_axis_name="c", subcore_axis_name="s", num_cores=_NC,
        num_subcores=_NS,
    )
    scratch = (
        [pltpu.VMEM((_CHUNK,), jnp.int32) for _ in range(_NBUF)]
        + [pltpu.VMEM((_CHUNK,), jnp.float32) for _ in range(_NBUF)]
        + [pltpu.VMEM((_CHUNK,), jnp.float32) for _ in range(_NBUF)]
        + [pltpu.VMEM_SHARED((_WSIZE,), jnp.float32)]
        + [pltpu.SemaphoreType.DMA for _ in range(3 * _NBUF)]
    )
    roast = pl.kernel(
        _roast_body,
        out_type=jax.ShapeDtypeStruct((_N,), jnp.float32),
        mesh=mesh,
        scratch_types=scratch,
    )
    out = roast(weight, IDX.reshape(-1), G.reshape(-1))
    return out.reshape(_NROW, _NCOL)


# speedup vs baseline: 865.2015x; 1.0119x over previous
import functools

import jax
import jax.numpy as jnp
from jax import lax
from jax.experimental import pallas as pl
from jax.experimental.pallas import tpu as pltpu
from jax.experimental.pallas import tpu_sc as plsc

_WSIZE = 1280000
_NROW, _NCOL = 100000, 128
_N = _NROW * _NCOL
_NC, _NS = 2, 16
_NW = _NC * _NS
_PER_W = _N // _NW
_NBUF = 4
_CHUNK = 4000
_NCHUNK = _PER_W // _CHUNK
_QUADS = _NCHUNK // _NBUF


def _roast_body(w_hbm, idx_hbm, g_hbm, out_hbm, *scratch):
    idx = scratch[0:_NBUF]
    g = scratch[_NBUF:2 * _NBUF]
    val = scratch[2 * _NBUF:3 * _NBUF]
    table = scratch[3 * _NBUF]
    sin = scratch[3 * _NBUF + 1:3 * _NBUF + 1 + _NBUF]
    sg = scratch[3 * _NBUF + 1 + _NBUF:3 * _NBUF + 1 + 2 * _NBUF]
    so = scratch[3 * _NBUF + 1 + 2 * _NBUF:3 * _NBUF + 1 + 3 * _NBUF]

    cid = lax.axis_index("c")
    sid = lax.axis_index("s")
    wid = sid * _NC + cid
    w0 = wid * _PER_W

    def issue_in(k, s):
        base = w0 + k * _CHUNK
        pltpu.async_copy(idx_hbm.at[pl.ds(base, _CHUNK)], idx[s], sin[s])
        pltpu.async_copy(g_hbm.at[pl.ds(base, _CHUNK)], g[s], sin[s])

    def wait_in(k, s):
        base = w0 + k * _CHUNK
        pltpu.make_async_copy(idx_hbm.at[pl.ds(base, _CHUNK)], idx[s], sin[s]).wait()
        pltpu.make_async_copy(g_hbm.at[pl.ds(base, _CHUNK)], g[s], sin[s]).wait()

    def issue_out(k, s):
        base = w0 + k * _CHUNK
        pltpu.async_copy(val[s], out_hbm.at[pl.ds(base, _CHUNK)], so[s])

    def wait_out(k, s):
        base = w0 + k * _CHUNK
        pltpu.make_async_copy(val[s], out_hbm.at[pl.ds(base, _CHUNK)], so[s]).wait()

    def issue_gather(s):
        pltpu.async_copy(table.at[idx[s]], val[s], sg[s])

    def wait_gather(s):
        pltpu.make_async_copy(table.at[idx[s]], val[s], sg[s]).wait()

    def multiply(s):
        val_v, g_v = val[s], g[s]

        @plsc.parallel_loop(0, _CHUNK, 16, unroll=8)
        def _(i):
            sl = pl.ds(i, 16)
            val_v[sl] = val_v[sl] * g_v[sl]

    for s in range(_NBUF):
        issue_in(s, s)
    seg = _WSIZE // _NS
    pltpu.sync_copy(
        w_hbm.at[pl.ds(sid * seg, seg)], table.at[pl.ds(sid * seg, seg)]
    )
    plsc.subcore_barrier()
    wait_in(0, 0)
    issue_gather(0)
    wait_in(1, 1)
    issue_gather(1)

    def quad_body(i, carry):
        for j in range(_NBUF):
            s = j
            k = _NBUF * i + j

            wait_gather(s)

            s2 = (j + 2) % _NBUF
            if j < 2:
                wait_in(k + 2, s2)

                @pl.when(i > 0)
                def _():
                    wait_out(k + 2 - _NBUF, s2)

                issue_gather(s2)
            else:

                @pl.when(i < _QUADS - 1)
                def _():
                    wait_in(k + 2, s2)
                    wait_out(k + 2 - _NBUF, s2)
                    issue_gather(s2)

            multiply(s)
            issue_out(k, s)

            @pl.when(i < _QUADS - 1)
            def _():
                issue_in(k + _NBUF, s)

        return carry

    lax.fori_loop(0, _QUADS, quad_body, 0)

    wait_out(_NCHUNK - 4, 0)
    wait_out(_NCHUNK - 3, 1)
    wait_out(_NCHUNK - 2, 2)
    wait_out(_NCHUNK - 1, 3)


def kernel(weight, IDX, G):
    mesh = plsc.VectorSubcoreMesh(
        core_axis_name="c", subcore_axis_name="s", num_cores=_NC,
        num_subcores=_NS,
    )
    scratch = (
        [pltpu.VMEM((_CHUNK,), jnp.int32) for _ in range(_NBUF)]
        + [pltpu.VMEM((_CHUNK,), jnp.float32) for _ in range(_NBUF)]
        + [pltpu.VMEM((_CHUNK,), jnp.float32) for _ in range(_NBUF)]
        + [pltpu.VMEM_SHARED((_WSIZE,), jnp.float32)]
        + [pltpu.SemaphoreType.DMA for _ in range(3 * _NBUF)]
    )
    roast = pl.kernel(
        _roast_body,
        out_type=jax.ShapeDtypeStruct((_N,), jnp.float32),
        mesh=mesh,
        scratch_types=scratch,
    )
    out = roast(weight, IDX.reshape(-1), G.reshape(-1))
    return out.reshape(_NROW, _NCOL)


# speedup vs baseline: 865.4570x; 1.0003x over previous
import functools

import jax
import jax.numpy as jnp
from jax import lax
from jax.experimental import pallas as pl
from jax.experimental.pallas import tpu as pltpu
from jax.experimental.pallas import tpu_sc as plsc

_WSIZE = 1280000
_NROW, _NCOL = 100000, 128
_N = _NROW * _NCOL
_NC, _NS = 2, 16
_NW = _NC * _NS
_PER_W = _N // _NW
_NBUF = 5
_CHUNK = 3200
_NCHUNK = _PER_W // _CHUNK
_QUADS = _NCHUNK // _NBUF


def _roast_body(w_hbm, idx_hbm, g_hbm, out_hbm, *scratch):
    idx = scratch[0:_NBUF]
    g = scratch[_NBUF:2 * _NBUF]
    val = scratch[2 * _NBUF:3 * _NBUF]
    table = scratch[3 * _NBUF]
    sin = scratch[3 * _NBUF + 1:3 * _NBUF + 1 + _NBUF]
    sg = scratch[3 * _NBUF + 1 + _NBUF:3 * _NBUF + 1 + 2 * _NBUF]
    so = scratch[3 * _NBUF + 1 + 2 * _NBUF:3 * _NBUF + 1 + 3 * _NBUF]

    cid = lax.axis_index("c")
    sid = lax.axis_index("s")
    wid = sid * _NC + cid
    w0 = wid * _PER_W

    def issue_in(k, s):
        base = w0 + k * _CHUNK
        pltpu.async_copy(idx_hbm.at[pl.ds(base, _CHUNK)], idx[s], sin[s])
        pltpu.async_copy(g_hbm.at[pl.ds(base, _CHUNK)], g[s], sin[s])

    def wait_in(k, s):
        base = w0 + k * _CHUNK
        pltpu.make_async_copy(idx_hbm.at[pl.ds(base, _CHUNK)], idx[s], sin[s]).wait()
        pltpu.make_async_copy(g_hbm.at[pl.ds(base, _CHUNK)], g[s], sin[s]).wait()

    def issue_out(k, s):
        base = w0 + k * _CHUNK
        pltpu.async_copy(val[s], out_hbm.at[pl.ds(base, _CHUNK)], so[s])

    def wait_out(k, s):
        base = w0 + k * _CHUNK
        pltpu.make_async_copy(val[s], out_hbm.at[pl.ds(base, _CHUNK)], so[s]).wait()

    def issue_gather(s):
        pltpu.async_copy(table.at[idx[s]], val[s], sg[s])

    def wait_gather(s):
        pltpu.make_async_copy(table.at[idx[s]], val[s], sg[s]).wait()

    def multiply(s):
        val_v, g_v = val[s], g[s]

        @plsc.parallel_loop(0, _CHUNK, 16, unroll=8)
        def _(i):
            sl = pl.ds(i, 16)
            val_v[sl] = val_v[sl] * g_v[sl]

    for s in range(_NBUF):
        issue_in(s, s)
    seg = _WSIZE // _NS
    pltpu.sync_copy(
        w_hbm.at[pl.ds(sid * seg, seg)], table.at[pl.ds(sid * seg, seg)]
    )
    plsc.subcore_barrier()
    wait_in(0, 0)
    issue_gather(0)
    wait_in(1, 1)
    issue_gather(1)

    def quad_body(i, carry):
        for j in range(_NBUF):
            s = j
            k = _NBUF * i + j

            wait_gather(s)

            s2 = (j + 2) % _NBUF
            if j < _NBUF - 2:
                wait_in(k + 2, s2)

                @pl.when(i > 0)
                def _():
                    wait_out(k + 2 - _NBUF, s2)

                issue_gather(s2)
            else:

                @pl.when(i < _QUADS - 1)
                def _():
                    wait_in(k + 2, s2)
                    wait_out(k + 2 - _NBUF, s2)
                    issue_gather(s2)

            multiply(s)
            issue_out(k, s)

            @pl.when(i < _QUADS - 1)
            def _():
                issue_in(k + _NBUF, s)

        return carry

    lax.fori_loop(0, _QUADS, quad_body, 0)

    for s in range(_NBUF):
        wait_out(_NCHUNK - _NBUF + s, s)


def kernel(weight, IDX, G):
    mesh = plsc.VectorSubcoreMesh(
        core_axis_name="c", subcore_axis_name="s", num_cores=_NC,
        num_subcores=_NS,
    )
    scratch = (
        [pltpu.VMEM((_CHUNK,), jnp.int32) for _ in range(_NBUF)]
        + [pltpu.VMEM((_CHUNK,), jnp.float32) for _ in range(_NBUF)]
        + [pltpu.VMEM((_CHUNK,), jnp.float32) for _ in range(_NBUF)]
        + [pltpu.VMEM_SHARED((_WSIZE,), jnp.float32)]
        + [pltpu.SemaphoreType.DMA for _ in range(3 * _NBUF)]
    )
    roast = pl.kernel(
        _roast_body,
        out_type=jax.ShapeDtypeStruct((_N,), jnp.float32),
        mesh=mesh,
        scratch_types=scratch,
    )
    out = roast(weight, IDX.reshape(-1), G.reshape(-1))
    return out.reshape(_NROW, _NCOL)


# speedup vs baseline: 866.1681x; 1.0008x over previous
import functools

import jax
import jax.numpy as jnp
from jax import lax
from jax.experimental import pallas as pl
from jax.experimental.pallas import tpu as pltpu
from jax.experimental.pallas import tpu_sc as plsc

_WSIZE = 1280000
_NROW, _NCOL = 100000, 128
_N = _NROW * _NCOL
_NC, _NS = 2, 16
_NW = _NC * _NS
_PER_W = _N // _NW
_NBUF = 5
_CHUNK = 3200
_NCHUNK = _PER_W // _CHUNK
_QUADS = _NCHUNK // _NBUF


def _roast_body(w_hbm, idx_hbm, g_hbm, out_hbm, *scratch):
    idx = scratch[0:_NBUF]
    g = scratch[_NBUF:2 * _NBUF]
    val = scratch[2 * _NBUF:3 * _NBUF]
    table = scratch[3 * _NBUF]
    sin = scratch[3 * _NBUF + 1:3 * _NBUF + 1 + _NBUF]
    sg = scratch[3 * _NBUF + 1 + _NBUF:3 * _NBUF + 1 + 2 * _NBUF]
    so = scratch[3 * _NBUF + 1 + 2 * _NBUF:3 * _NBUF + 1 + 3 * _NBUF]

    cid = lax.axis_index("c")
    sid = lax.axis_index("s")
    wid = sid * _NC + cid
    w0 = wid * _PER_W

    def issue_in_idx(k, s):
        base = w0 + k * _CHUNK
        pltpu.async_copy(idx_hbm.at[pl.ds(base, _CHUNK)], idx[s], sin[s])

    def issue_in_g(k, s):
        base = w0 + k * _CHUNK
        pltpu.async_copy(g_hbm.at[pl.ds(base, _CHUNK)], g[s], sin[s])

    def issue_in(k, s):
        issue_in_idx(k, s)
        issue_in_g(k, s)

    def wait_in(k, s):
        base = w0 + k * _CHUNK
        pltpu.make_async_copy(idx_hbm.at[pl.ds(base, _CHUNK)], idx[s], sin[s]).wait()
        pltpu.make_async_copy(g_hbm.at[pl.ds(base, _CHUNK)], g[s], sin[s]).wait()

    def issue_out(k, s):
        base = w0 + k * _CHUNK
        pltpu.async_copy(val[s], out_hbm.at[pl.ds(base, _CHUNK)], so[s])

    def wait_out(k, s):
        base = w0 + k * _CHUNK
        pltpu.make_async_copy(val[s], out_hbm.at[pl.ds(base, _CHUNK)], so[s]).wait()

    def issue_gather(s):
        pltpu.async_copy(table.at[idx[s]], val[s], sg[s])

    def wait_gather(s):
        pltpu.make_async_copy(table.at[idx[s]], val[s], sg[s]).wait()

    def multiply(s):
        val_v, g_v = val[s], g[s]

        @plsc.parallel_loop(0, _CHUNK, 16, unroll=8)
        def _(i):
            sl = pl.ds(i, 16)
            val_v[sl] = val_v[sl] * g_v[sl]

    for s in range(_NBUF):
        issue_in(s, s)
    seg = _WSIZE // _NS
    pltpu.sync_copy(
        w_hbm.at[pl.ds(sid * seg, seg)], table.at[pl.ds(sid * seg, seg)]
    )
    plsc.subcore_barrier()
    wait_in(0, 0)
    issue_gather(0)
    wait_in(1, 1)
    issue_gather(1)

    def quad_body(i, carry):
        for j in range(_NBUF):
            s = j
            k = _NBUF * i + j

            wait_gather(s)

            @pl.when(i < _QUADS - 1)
            def _():
                issue_in_idx(k + _NBUF, s)

            s2 = (j + 2) % _NBUF
            if j < _NBUF - 2:
                wait_in(k + 2, s2)

                @pl.when(i > 0)
                def _():
                    wait_out(k + 2 - _NBUF, s2)

                issue_gather(s2)
            else:

                @pl.when(i < _QUADS - 1)
                def _():
                    wait_in(k + 2, s2)
                    wait_out(k + 2 - _NBUF, s2)
                    issue_gather(s2)

            multiply(s)
            issue_out(k, s)

            @pl.when(i < _QUADS - 1)
            def _():
                issue_in_g(k + _NBUF, s)

        return carry

    lax.fori_loop(0, _QUADS, quad_body, 0)

    for s in range(_NBUF):
        wait_out(_NCHUNK - _NBUF + s, s)


def kernel(weight, IDX, G):
    mesh = plsc.VectorSubcoreMesh(
        core_axis_name="c", subcore_axis_name="s", num_cores=_NC,
        num_subcores=_NS,
    )
    scratch = (
        [pltpu.VMEM((_CHUNK,), jnp.int32) for _ in range(_NBUF)]
        + [pltpu.VMEM((_CHUNK,), jnp.float32) for _ in range(_NBUF)]
        + [pltpu.VMEM((_CHUNK,), jnp.float32) for _ in range(_NBUF)]
        + [pltpu.VMEM_SHARED((_WSIZE,), jnp.float32)]
        + [pltpu.SemaphoreType.DMA for _ in range(3 * _NBUF)]
    )
    roast = pl.kernel(
        _roast_body,
        out_type=jax.ShapeDtypeStruct((_N,), jnp.float32),
        mesh=mesh,
        scratch_types=scratch,
    )
    out = roast(weight, IDX.reshape(-1), G.reshape(-1))
    return out.reshape(_NROW, _NCOL)
